# Initial kernel scaffold; baseline (speedup 1.0000x reference)
import jax
import jax.numpy as jnp
from jax.experimental import pallas as pl


def kernel(user_id, time, device, item_id, category, user_table, time_table, device_table, item_table, category_table, W_user, b_user, W_item, b_item):
    raise NotImplementedError("write your pallas kernel here")


# speedup vs baseline: 4.6956x; 4.6956x over previous
import functools

import jax
import jax.numpy as jnp
from jax import lax
from jax.experimental import pallas as pl
from jax.experimental.pallas import tpu as pltpu
from jax.experimental.pallas import tpu_sc as plsc

BATCH = 16384
D = 128
NUM_CORES = 2
NUM_SUBCORES = 16
NUM_WORKERS = NUM_CORES * NUM_SUBCORES
CHUNK = 128
NBUF = 4
ROWS = BATCH
CPW = ROWS // NUM_WORKERS // CHUNK
TB = 2048


def _sc_gather_body(u_tab, t_tab, d_tab, i_tab, c_tab,
                    u_idx, t_idx, d_idx, i_idx, c_idx,
                    out, idx_vs, bufs, small_tabs, isem, gsems, wsems):
  wid = lax.axis_index("s") * NUM_CORES + lax.axis_index("c")
  base = wid * (CPW * CHUNK)
  crow0 = wid * CPW

  @pl.when(lax.axis_index("s") == 0)
  def _():
    for src, dst in zip((t_tab, d_tab, c_tab), small_tabs):
      pltpu.sync_copy(src, dst)
  plsc.subcore_barrier()

  tables = [u_tab, small_tabs[0], small_tabs[1], i_tab, small_tabs[2]]
  idxs = [u_idx, t_idx, d_idx, i_idx, c_idx]

  icopies = []
  for k in range(5):
    cp = pltpu.make_async_copy(idxs[k].at[pl.ds(crow0, CPW)], idx_vs[k], isem)
    cp.start()
    icopies.append(cp)
  for cp in icopies:
    cp.wait()

  work = []
  for k in range(5):
    for ci in range(CPW):
      work.append((tables[k], idx_vs[k], ci, k * ROWS))
  n = len(work)

  def start_write(c):
    _, _, ci, off = work[c]
    w = pltpu.make_async_copy(bufs[c % NBUF],
                              out.at[pl.ds(off + base + ci * CHUNK, CHUNK)],
                              wsems.at[c % NBUF])
    w.start()
    return w

  gathers = [None] * n
  writes = [None] * n
  for c in range(n):
    tab, idxv, ci, _ = work[c]
    if c >= NBUF:
      writes[c - NBUF].wait()
    cp = pltpu.make_async_copy(tab.at[idxv.at[ci]], bufs[c % NBUF],
                               gsems.at[c % NBUF])
    cp.start()
    gathers[c] = cp
    if c >= 1:
      gathers[c - 1].wait()
      writes[c - 1] = start_write(c - 1)
  gathers[n - 1].wait()
  writes[n - 1] = start_write(n - 1)
  for c in range(max(0, n - NBUF), n):
    writes[c].wait()


def _sc_gather(u_tab, t_tab, d_tab, i_tab, c_tab,
               u_idx, t_idx, d_idx, i_idx, c_idx):
  mesh = plsc.VectorSubcoreMesh(core_axis_name="c", subcore_axis_name="s")
  out = jax.ShapeDtypeStruct((5 * ROWS, D), jnp.float32)
  kern = pl.kernel(
      _sc_gather_body,
      out_type=out,
      mesh=mesh,
      scratch_types=[
          [pltpu.VMEM((CPW, CHUNK), jnp.int32) for _ in range(5)],
          [pltpu.VMEM((CHUNK, D), jnp.float32) for _ in range(NBUF)],
          [pltpu.VMEM_SHARED((1000, D), jnp.float32) for _ in range(3)],
          pltpu.SemaphoreType.DMA,
          pltpu.SemaphoreType.DMA((NBUF,)),
          pltpu.SemaphoreType.DMA((NBUF,)),
      ],
  )
  return kern(u_tab, t_tab, d_tab, i_tab, c_tab,
              u_idx, t_idx, d_idx, i_idx, c_idx)


def _tc_body(gu, gt, gd, gi, gc, wu, bu, wi, bi, o):
  xu = jnp.concatenate([gu[...], gt[...], gd[...]], axis=1)
  u = jnp.dot(xu, wu[...], preferred_element_type=jnp.float32) + bu[...]
  xi = jnp.concatenate([gi[...], gc[...]], axis=1)
  v = jnp.dot(xi, wi[...], preferred_element_type=jnp.float32) + bi[...]
  ones = jnp.ones((D, 1), jnp.float32)
  o[...] = jnp.dot(u * v, ones, preferred_element_type=jnp.float32)[:, 0]


def _tc_combine(fused, W_user, b_user, W_item, b_item):
  grid = (ROWS // TB,)
  nb = ROWS // TB

  def view(k):
    return pl.BlockSpec((TB, D), lambda i, k=k: (k * nb + i, 0))

  return pl.pallas_call(
      _tc_body,
      grid=grid,
      in_specs=[
          view(0), view(1), view(2), view(3), view(4),
          pl.BlockSpec((3 * D, D), lambda i: (0, 0)),
          pl.BlockSpec((D,), lambda i: (0,)),
          pl.BlockSpec((2 * D, D), lambda i: (0, 0)),
          pl.BlockSpec((D,), lambda i: (0,)),
      ],
      out_specs=pl.BlockSpec((TB,), lambda i: (i,)),
      out_shape=jax.ShapeDtypeStruct((ROWS,), jnp.float32),
  )(fused, fused, fused, fused, fused,
    W_user, b_user, W_item, b_item)


@jax.jit
def kernel(user_id, time, device, item_id, category,
           user_table, time_table, device_table, item_table, category_table,
           W_user, b_user, W_item, b_item):
  def r2(x):
    return x.astype(jnp.int32).reshape(BATCH // CHUNK, CHUNK)
  idx_all = [r2(user_id), r2(time), r2(device), r2(item_id), r2(category)]
  tabs = (user_table, time_table, device_table, item_table, category_table)

  fused = _sc_gather(*tabs, *idx_all)
  return _tc_combine(fused, W_user, b_user, W_item, b_item)


# speedup vs baseline: 4.6965x; 1.0002x over previous
import functools

import jax
import jax.numpy as jnp
from jax import lax
from jax.experimental import pallas as pl
from jax.experimental.pallas import tpu as pltpu
from jax.experimental.pallas import tpu_sc as plsc

BATCH = 16384
D = 128
NUM_CORES = 2
NUM_SUBCORES = 16
NUM_WORKERS = NUM_CORES * NUM_SUBCORES
CHUNK = 128
NBUF = 4
ROWS = BATCH
CPW = ROWS // NUM_WORKERS // CHUNK
TB = 4096


def _sc_gather_body(u_tab, t_tab, d_tab, i_tab, c_tab,
                    u_idx, t_idx, d_idx, i_idx, c_idx,
                    out, idx_vs, bufs, small_tabs, isem, gsems, wsems):
  wid = lax.axis_index("s") * NUM_CORES + lax.axis_index("c")
  base = wid * (CPW * CHUNK)
  crow0 = wid * CPW

  @pl.when(lax.axis_index("s") == 0)
  def _():
    for src, dst in zip((t_tab, d_tab, c_tab), small_tabs):
      pltpu.sync_copy(src, dst)
  plsc.subcore_barrier()

  tables = [u_tab, small_tabs[0], small_tabs[1], i_tab, small_tabs[2]]
  idxs = [u_idx, t_idx, d_idx, i_idx, c_idx]

  icopies = []
  for k in range(5):
    cp = pltpu.make_async_copy(idxs[k].at[pl.ds(crow0, CPW)], idx_vs[k], isem)
    cp.start()
    icopies.append(cp)
  for cp in icopies:
    cp.wait()

  work = []
  for k in range(5):
    for ci in range(CPW):
      work.append((tables[k], idx_vs[k], ci, k * ROWS))
  n = len(work)

  def start_write(c):
    _, _, ci, off = work[c]
    w = pltpu.make_async_copy(bufs[c % NBUF],
                              out.at[pl.ds(off + base + ci * CHUNK, CHUNK)],
                              wsems.at[c % NBUF])
    w.start()
    return w

  gathers = [None] * n
  writes = [None] * n
  for c in range(n):
    tab, idxv, ci, _ = work[c]
    if c >= NBUF:
      writes[c - NBUF].wait()
    cp = pltpu.make_async_copy(tab.at[idxv.at[ci]], bufs[c % NBUF],
                               gsems.at[c % NBUF])
    cp.start()
    gathers[c] = cp
    if c >= 1:
      gathers[c - 1].wait()
      writes[c - 1] = start_write(c - 1)
  gathers[n - 1].wait()
  writes[n - 1] = start_write(n - 1)
  for c in range(max(0, n - NBUF), n):
    writes[c].wait()


def _sc_gather(u_tab, t_tab, d_tab, i_tab, c_tab,
               u_idx, t_idx, d_idx, i_idx, c_idx):
  mesh = plsc.VectorSubcoreMesh(core_axis_name="c", subcore_axis_name="s")
  out = jax.ShapeDtypeStruct((5 * ROWS, D), jnp.float32)
  kern = pl.kernel(
      _sc_gather_body,
      out_type=out,
      mesh=mesh,
      scratch_types=[
          [pltpu.VMEM((CPW, CHUNK), jnp.int32) for _ in range(5)],
          [pltpu.VMEM((CHUNK, D), jnp.float32) for _ in range(NBUF)],
          [pltpu.VMEM_SHARED((1000, D), jnp.float32) for _ in range(3)],
          pltpu.SemaphoreType.DMA,
          pltpu.SemaphoreType.DMA((NBUF,)),
          pltpu.SemaphoreType.DMA((NBUF,)),
      ],
  )
  return kern(u_tab, t_tab, d_tab, i_tab, c_tab,
              u_idx, t_idx, d_idx, i_idx, c_idx)


def _tc_body(gu, gt, gd, gi, gc, wu, bu, wi, bi, o):
  xu = jnp.concatenate([gu[...], gt[...], gd[...]], axis=1)
  u = jnp.dot(xu, wu[...], preferred_element_type=jnp.float32) + bu[...]
  xi = jnp.concatenate([gi[...], gc[...]], axis=1)
  v = jnp.dot(xi, wi[...], preferred_element_type=jnp.float32) + bi[...]
  ones = jnp.ones((D, 1), jnp.float32)
  o[...] = jnp.dot(u * v, ones, preferred_element_type=jnp.float32)[:, 0]


def _tc_combine(fused, W_user, b_user, W_item, b_item):
  grid = (ROWS // TB,)
  nb = ROWS // TB

  def view(k):
    return pl.BlockSpec((TB, D), lambda i, k=k: (k * nb + i, 0))

  return pl.pallas_call(
      _tc_body,
      grid=grid,
      in_specs=[
          view(0), view(1), view(2), view(3), view(4),
          pl.BlockSpec((3 * D, D), lambda i: (0, 0)),
          pl.BlockSpec((D,), lambda i: (0,)),
          pl.BlockSpec((2 * D, D), lambda i: (0, 0)),
          pl.BlockSpec((D,), lambda i: (0,)),
      ],
      out_specs=pl.BlockSpec((TB,), lambda i: (i,)),
      out_shape=jax.ShapeDtypeStruct((ROWS,), jnp.float32),
  )(fused, fused, fused, fused, fused,
    W_user, b_user, W_item, b_item)


@jax.jit
def kernel(user_id, time, device, item_id, category,
           user_table, time_table, device_table, item_table, category_table,
           W_user, b_user, W_item, b_item):
  def r2(x):
    return x.astype(jnp.int32).reshape(BATCH // CHUNK, CHUNK)
  idx_all = [r2(user_id), r2(time), r2(device), r2(item_id), r2(category)]
  tabs = (user_table, time_table, device_table, item_table, category_table)

  fused = _sc_gather(*tabs, *idx_all)
  return _tc_combine(fused, W_user, b_user, W_item, b_item)


# speedup vs baseline: 4.9950x; 1.0636x over previous
import functools

import jax
import jax.numpy as jnp
from jax import lax
from jax.experimental import pallas as pl
from jax.experimental.pallas import tpu as pltpu
from jax.experimental.pallas import tpu_sc as plsc

BATCH = 16384
D = 128
NUM_CORES = 2
NUM_SUBCORES = 16
NUM_WORKERS = NUM_CORES * NUM_SUBCORES
CHUNK = 128
NBUF = 4
ROWS = BATCH
CPW = ROWS // NUM_WORKERS // CHUNK
TB = 4096


def _sc_gather_body(u_tab, t_tab, d_tab, i_tab, c_tab,
                    u_idx, t_idx, d_idx, i_idx, c_idx,
                    out, idx_vs, bufs, small_tabs, isem, ssem, gsems, wsems):
  wid = lax.axis_index("s") * NUM_CORES + lax.axis_index("c")
  base = wid * (CPW * CHUNK)
  crow0 = wid * CPW

  sid = lax.axis_index("s")
  idxs = [u_idx, t_idx, d_idx, i_idx, c_idx]

  icopies = []
  for k in range(5):
    cp = pltpu.make_async_copy(idxs[k].at[pl.ds(crow0, CPW)], idx_vs[k], isem)
    cp.start()
    icopies.append(cp)

  scopies = [pltpu.make_async_copy(src, dst, ssem)
             for src, dst in zip((t_tab, d_tab, c_tab), small_tabs)]

  @pl.when(sid == 0)
  def _():
    for cp in scopies:
      cp.start()

  for cp in icopies:
    cp.wait()

  tables = [u_tab, i_tab, small_tabs[0], small_tabs[1], small_tabs[2]]
  offs = [0, 3 * ROWS, 1 * ROWS, 2 * ROWS, 4 * ROWS]
  kidx = [0, 3, 1, 2, 4]
  work = []
  for k in range(5):
    for ci in range(CPW):
      work.append((tables[k], idx_vs[kidx[k]], ci, offs[k]))
  n = len(work)
  phase2 = 2 * CPW

  def start_write(c):
    _, _, ci, off = work[c]
    w = pltpu.make_async_copy(bufs[c % NBUF],
                              out.at[pl.ds(off + base + ci * CHUNK, CHUNK)],
                              wsems.at[c % NBUF])
    w.start()
    return w

  gathers = [None] * n
  writes = [None] * n
  for c in range(n):
    if c == phase2:
      @pl.when(sid == 0)
      def _():
        for cp in scopies:
          cp.wait()
      plsc.subcore_barrier()
    tab, idxv, ci, _ = work[c]
    if c >= NBUF:
      writes[c - NBUF].wait()
    cp = pltpu.make_async_copy(tab.at[idxv.at[ci]], bufs[c % NBUF],
                               gsems.at[c % NBUF])
    cp.start()
    gathers[c] = cp
    if c >= 1:
      gathers[c - 1].wait()
      writes[c - 1] = start_write(c - 1)
  gathers[n - 1].wait()
  writes[n - 1] = start_write(n - 1)
  for c in range(max(0, n - NBUF), n):
    writes[c].wait()


def _sc_gather(u_tab, t_tab, d_tab, i_tab, c_tab,
               u_idx, t_idx, d_idx, i_idx, c_idx):
  mesh = plsc.VectorSubcoreMesh(core_axis_name="c", subcore_axis_name="s")
  out = jax.ShapeDtypeStruct((5 * ROWS, D), jnp.float32)
  kern = pl.kernel(
      _sc_gather_body,
      out_type=out,
      mesh=mesh,
      scratch_types=[
          [pltpu.VMEM((CPW, CHUNK), jnp.int32) for _ in range(5)],
          [pltpu.VMEM((CHUNK, D), jnp.float32) for _ in range(NBUF)],
          [pltpu.VMEM_SHARED((1000, D), jnp.float32) for _ in range(3)],
          pltpu.SemaphoreType.DMA,
          pltpu.SemaphoreType.DMA,
          pltpu.SemaphoreType.DMA((NBUF,)),
          pltpu.SemaphoreType.DMA((NBUF,)),
      ],
  )
  return kern(u_tab, t_tab, d_tab, i_tab, c_tab,
              u_idx, t_idx, d_idx, i_idx, c_idx)


def _tc_body(gu, gt, gd, gi, gc, wu, bu, wi, bi, o):
  xu = jnp.concatenate([gu[...], gt[...], gd[...]], axis=1)
  u = jnp.dot(xu, wu[...], preferred_element_type=jnp.float32) + bu[...]
  xi = jnp.concatenate([gi[...], gc[...]], axis=1)
  v = jnp.dot(xi, wi[...], preferred_element_type=jnp.float32) + bi[...]
  ones = jnp.ones((D, 1), jnp.float32)
  o[...] = jnp.dot(u * v, ones, preferred_element_type=jnp.float32)[:, 0]


def _tc_combine(fused, W_user, b_user, W_item, b_item):
  grid = (ROWS // TB,)
  nb = ROWS // TB

  def view(k):
    return pl.BlockSpec((TB, D), lambda i, k=k: (k * nb + i, 0))

  return pl.pallas_call(
      _tc_body,
      grid=grid,
      in_specs=[
          view(0), view(1), view(2), view(3), view(4),
          pl.BlockSpec((3 * D, D), lambda i: (0, 0)),
          pl.BlockSpec((D,), lambda i: (0,)),
          pl.BlockSpec((2 * D, D), lambda i: (0, 0)),
          pl.BlockSpec((D,), lambda i: (0,)),
      ],
      out_specs=pl.BlockSpec((TB,), lambda i: (i,)),
      out_shape=jax.ShapeDtypeStruct((ROWS,), jnp.float32),
  )(fused, fused, fused, fused, fused,
    W_user, b_user, W_item, b_item)


@jax.jit
def kernel(user_id, time, device, item_id, category,
           user_table, time_table, device_table, item_table, category_table,
           W_user, b_user, W_item, b_item):
  def r2(x):
    return x.astype(jnp.int32).reshape(BATCH // CHUNK, CHUNK)
  idx_all = [r2(user_id), r2(time), r2(device), r2(item_id), r2(category)]
  tabs = (user_table, time_table, device_table, item_table, category_table)

  fused = _sc_gather(*tabs, *idx_all)
  return _tc_combine(fused, W_user, b_user, W_item, b_item)
